# baseline (speedup 1.0000x reference)
import functools

import jax
import jax.numpy as jnp
from jax.experimental import pallas as pl
from jax.experimental.pallas import tpu as pltpu
from jax.experimental.pallas import tpu_sc as plsc

B = 8
C = 3
H = 352
N = 484
NPAD = 512
D = 768
K = 9
NEG = -3e38


def _band_matmul(src_b, band_ref):
    acc = jnp.zeros((H, C * H), jnp.float32)
    for ci in range(C):
        for dh in range(3):
            lhs = src_b[ci, dh:dh + H, :]
            rhs = band_ref[ci * 3 + dh]
            acc = acc + jnp.dot(lhs, rhs, preferred_element_type=jnp.float32)
    return acc


def _conv_patch_kernel(b0_ref, b1_ref, band0_ref, band1_ref, x_ref, amp_ref,
                       p_ref, xs, hs):
    @pl.when(pl.program_id(0) == 0)
    def _init():
        xs[...] = jnp.zeros_like(xs)
        hs[...] = jnp.zeros_like(hs)

    xs[:, 1:H + 1, 1:H + 1] = x_ref[0]
    xp = xs[...]
    y0 = _band_matmul(xp.astype(jnp.bfloat16), band0_ref)
    for co in range(C):
        hco = jnp.maximum(y0[:, co * H:(co + 1) * H] + b0_ref[co], 0.0)
        hs[co, 1:H + 1, 1:H + 1] = hco
    y1 = _band_matmul(hs[...].astype(jnp.bfloat16), band1_ref)
    prs = []
    for co in range(C):
        pr = y1[:, co * H:(co + 1) * H] + b1_ref[co]
        amp_ref[0, co] = pr * xp[co, 1:H + 1, 1:H + 1]
        prs.append(pr)
    v = jnp.stack(prs)
    v = v.reshape(C, 22, 16, 22, 16)
    v = jnp.transpose(v, (1, 3, 0, 2, 4))
    v = v.reshape(N, D)
    p_ref[0] = jnp.concatenate([v, jnp.zeros((NPAD - N, D), jnp.float32)], axis=0)


def _make_band(W):
    jp = jnp.arange(H + 2)
    j = jnp.arange(H)
    diff = jp[:, None] - j[None, :]
    vals = jnp.zeros((C, C, 3, H + 2, H), jnp.float32)
    for dw in range(3):
        mask = (diff == dw).astype(jnp.float32)
        vals = vals + W[:, :, :, dw][..., None, None] * mask
    vals = vals.transpose(1, 2, 3, 0, 4)
    return vals.reshape(C * 3, H + 2, C * H).astype(jnp.bfloat16)


def _vig_a_kernel(p_ref, we_ref, be_ref, feat_ref, idx_ref):
    pb = p_ref[0].astype(jnp.bfloat16)
    feat = jnp.dot(pb, we_ref[...], preferred_element_type=jnp.float32)
    feat = feat + be_ref[...]
    feat_ref[0] = feat
    sq = jnp.sum(feat * feat, axis=1, keepdims=True)
    fb = feat.astype(jnp.bfloat16)
    gram = jax.lax.dot_general(fb, fb, (((1,), (1,)), ((), ())),
                               preferred_element_type=jnp.float32)
    dist = sq + sq.T - 2.0 * gram
    col = jax.lax.broadcasted_iota(jnp.int32, (NPAD, NPAD), 1)
    dist = jnp.where(col < N, dist, jnp.inf)
    base = pl.program_id(0) * NPAD
    col128 = jax.lax.broadcasted_iota(jnp.int32, (NPAD, 128), 1)
    idxm = jnp.zeros((NPAD, 128), jnp.int32)
    sel0 = None
    for t in range(K):
        rowmin = jnp.min(dist, axis=1, keepdims=True)
        cand = jnp.where(dist == rowmin, col, NPAD)
        sel = jnp.min(cand, axis=1, keepdims=True)
        idxm = idxm + jnp.where(col128 == t, sel + base, 0)
        if sel0 is None:
            sel0 = sel
        dist = jnp.where(col == sel, jnp.inf, dist)
    idxm = jnp.where((col128 >= K) & (col128 < 16), sel0 + base, idxm)
    idx_ref[0] = idxm


def _vig_b_kernel(feat_ref, mr_ref, wgt_ref, wgb_ref, bg_ref, out_ref):
    feat = feat_ref[0]
    fb = feat.astype(jnp.bfloat16)
    hh = jnp.dot(fb, wgt_ref[...], preferred_element_type=jnp.float32)
    hh = hh + jnp.dot(mr_ref[0].astype(jnp.bfloat16), wgb_ref[...],
                      preferred_element_type=jnp.float32)
    hh = hh + bg_ref[...]
    out_ref[0] = feat + jnp.maximum(hh, 0.0)


NTILES = 32
CH = 4


def _sc_gather_kernel(feat_hbm, idx_hbm, mr_hbm, idxv, fbuf, gbuf, obuf, sem):
    nc = 2
    wid = jax.lax.axis_index("s") * nc + jax.lax.axis_index("c")
    nrows = feat_hbm.shape[0]
    rows_per_tile = nrows // NTILES
    tile_base = wid * rows_per_tile
    nchunks = rows_per_tile // CH

    def fire(slot, g):
        cbase = tile_base + g * CH
        pltpu.sync_copy(idx_hbm.at[pl.ds(cbase, CH)], idxv.at[slot])
        pltpu.sync_copy(feat_hbm.at[pl.ds(cbase, CH)], fbuf.at[slot])
        for t in range(CH):
            pltpu.async_copy(
                feat_hbm.at[idxv.at[slot, t, pl.ds(0, 16)]],
                gbuf.at[slot, t], sem)

    fire(0, 0)

    def chunk(g, carry):
        slot = jax.lax.rem(g, 2)
        nslot = jax.lax.rem(g + 1, 2)

        @pl.when(g + 1 < nchunks)
        def _prefetch():
            fire(nslot, g + 1)

        for t in range(CH):
            pltpu.make_async_copy(
                feat_hbm.at[idxv.at[slot, t, pl.ds(0, 16)]],
                gbuf.at[slot, t], sem).wait()
        def tbody(t, tc):
            for dd in range(D // 16):
                sl = pl.ds(dd * 16, 16)
                m = gbuf[slot, t, 0, sl]
                for r in range(1, 16):
                    m = jnp.maximum(m, gbuf[slot, t, r, sl])
                obuf[t, sl] = m - fbuf[slot, t, sl]
            return tc
        jax.lax.fori_loop(0, CH, tbody, 0)
        cbase = tile_base + g * CH
        pltpu.sync_copy(obuf, mr_hbm.at[pl.ds(cbase, CH)])
        return carry

    jax.lax.fori_loop(0, nchunks, chunk, 0)


def kernel(x, W0, b0, W1, b1, We, be, Wg, bg):
    band0 = _make_band(W0)
    band1 = _make_band(W1)

    amp_src, p = pl.pallas_call(
        _conv_patch_kernel,
        grid=(B,),
        in_specs=[
            pl.BlockSpec(memory_space=pltpu.SMEM),
            pl.BlockSpec(memory_space=pltpu.SMEM),
            pl.BlockSpec((C * 3, H + 2, C * H), lambda i: (0, 0, 0)),
            pl.BlockSpec((C * 3, H + 2, C * H), lambda i: (0, 0, 0)),
            pl.BlockSpec((1, C, H, H), lambda i: (i, 0, 0, 0)),
        ],
        out_specs=[
            pl.BlockSpec((1, C, H, H), lambda i: (i, 0, 0, 0)),
            pl.BlockSpec((1, NPAD, D), lambda i: (i, 0, 0)),
        ],
        out_shape=[
            jax.ShapeDtypeStruct((B, C, H, H), jnp.float32),
            jax.ShapeDtypeStruct((B, NPAD, D), jnp.float32),
        ],
        scratch_shapes=[
            pltpu.VMEM((C, H + 2, H + 2), jnp.float32),
            pltpu.VMEM((C, H + 2, H + 2), jnp.float32),
        ],
    )(b0, b1, band0, band1, x)

    web = We.astype(jnp.bfloat16)
    wgtb = Wg[:D].astype(jnp.bfloat16)
    wgbb = Wg[D:].astype(jnp.bfloat16)
    be2 = be.reshape(1, D)
    bg2 = bg.reshape(1, D)

    HB = B
    HROWS = HB * NPAD
    sc_gather = functools.partial(
        pl.kernel,
        mesh=plsc.VectorSubcoreMesh(core_axis_name="c", subcore_axis_name="s"),
        out_type=jax.ShapeDtypeStruct((HROWS, D), jnp.float32),
        scratch_types=[
            pltpu.VMEM((2, CH, 128), jnp.int32),
            pltpu.VMEM((2, CH, D), jnp.float32),
            pltpu.VMEM((2, CH, 16, D), jnp.float32),
            pltpu.VMEM((CH, D), jnp.float32),
            pltpu.SemaphoreType.DMA,
        ],
    )(_sc_gather_kernel)

    halves = []
    for hb in range(1):
        ph = jax.lax.slice_in_dim(p, hb * HB, (hb + 1) * HB, axis=0)
        feat, idx = pl.pallas_call(
            _vig_a_kernel,
            grid=(HB,),
            in_specs=[
                pl.BlockSpec((1, NPAD, D), lambda i: (i, 0, 0)),
                pl.BlockSpec((D, D), lambda i: (0, 0)),
                pl.BlockSpec((1, D), lambda i: (0, 0)),
            ],
            out_specs=[
                pl.BlockSpec((1, NPAD, D), lambda i: (i, 0, 0)),
                pl.BlockSpec((1, NPAD, 128), lambda i: (i, 0, 0)),
            ],
            out_shape=[
                jax.ShapeDtypeStruct((HB, NPAD, D), jnp.float32),
                jax.ShapeDtypeStruct((HB, NPAD, 128), jnp.int32),
            ],
        )(ph, web, be2)
        mr = sc_gather(feat.reshape(HROWS, D), idx.reshape(HROWS, 128))
        mr = mr.reshape(HB, NPAD, D)
        out = pl.pallas_call(
            _vig_b_kernel,
            grid=(HB,),
            in_specs=[
                pl.BlockSpec((1, NPAD, D), lambda i: (i, 0, 0)),
                pl.BlockSpec((1, NPAD, D), lambda i: (i, 0, 0)),
                pl.BlockSpec((D, D), lambda i: (0, 0)),
                pl.BlockSpec((D, D), lambda i: (0, 0)),
                pl.BlockSpec((1, D), lambda i: (0, 0)),
            ],
            out_specs=pl.BlockSpec((1, NPAD, D), lambda i: (i, 0, 0)),
            out_shape=jax.ShapeDtypeStruct((HB, NPAD, D), jnp.float32),
        )(feat, mr, wgtb, wgbb, bg2)
        halves.append(out[:, :N, :])

    amp_low = jnp.concatenate(halves, axis=0)
    return (amp_src, amp_low)


# speedup vs baseline: 1.0008x; 1.0008x over previous
import functools

import jax
import jax.numpy as jnp
from jax.experimental import pallas as pl
from jax.experimental.pallas import tpu as pltpu
from jax.experimental.pallas import tpu_sc as plsc

B = 8
C = 3
H = 352
N = 484
NPAD = 512
D = 768
K = 9
NEG = -3e38


def _band_matmul(src_b, band_ref):
    acc = jnp.zeros((H, C * H), jnp.float32)
    for ci in range(C):
        for dh in range(3):
            lhs = src_b[ci, dh:dh + H, :]
            rhs = band_ref[ci * 3 + dh]
            acc = acc + jnp.dot(lhs, rhs, preferred_element_type=jnp.float32)
    return acc


def _conv_patch_kernel(b0_ref, b1_ref, band0_ref, band1_ref, x_ref, amp_ref,
                       p_ref, xs, hs):
    @pl.when(pl.program_id(0) == 0)
    def _init():
        xs[...] = jnp.zeros_like(xs)
        hs[...] = jnp.zeros_like(hs)

    xs[:, 1:H + 1, 1:H + 1] = x_ref[0]
    xp = xs[...]
    y0 = _band_matmul(xp.astype(jnp.bfloat16), band0_ref)
    for co in range(C):
        hco = jnp.maximum(y0[:, co * H:(co + 1) * H] + b0_ref[co], 0.0)
        hs[co, 1:H + 1, 1:H + 1] = hco
    y1 = _band_matmul(hs[...].astype(jnp.bfloat16), band1_ref)
    prs = []
    for co in range(C):
        pr = y1[:, co * H:(co + 1) * H] + b1_ref[co]
        amp_ref[0, co] = pr * xp[co, 1:H + 1, 1:H + 1]
        prs.append(pr)
    v = jnp.stack(prs)
    v = v.reshape(C, 22, 16, 22, 16)
    v = jnp.transpose(v, (1, 3, 0, 2, 4))
    v = v.reshape(N, D)
    p_ref[0] = jnp.concatenate([v, jnp.zeros((NPAD - N, D), jnp.float32)], axis=0)


def _make_band(W):
    jp = jnp.arange(H + 2)
    j = jnp.arange(H)
    diff = jp[:, None] - j[None, :]
    vals = jnp.zeros((C, C, 3, H + 2, H), jnp.float32)
    for dw in range(3):
        mask = (diff == dw).astype(jnp.float32)
        vals = vals + W[:, :, :, dw][..., None, None] * mask
    vals = vals.transpose(1, 2, 3, 0, 4)
    return vals.reshape(C * 3, H + 2, C * H).astype(jnp.bfloat16)


def _vig_a_kernel(p_ref, we_ref, be_ref, feat_ref, idx_ref):
    pb = p_ref[0].astype(jnp.bfloat16)
    feat = jnp.dot(pb, we_ref[...], preferred_element_type=jnp.float32)
    feat = feat + be_ref[...]
    feat_ref[0] = feat
    sq = jnp.sum(feat * feat, axis=1, keepdims=True)
    fb = feat.astype(jnp.bfloat16)
    gram = jax.lax.dot_general(fb, fb, (((1,), (1,)), ((), ())),
                               preferred_element_type=jnp.float32)
    dist = sq + sq.T - 2.0 * gram
    col = jax.lax.broadcasted_iota(jnp.int32, (NPAD, NPAD), 1)
    dist = jnp.where(col < N, dist, jnp.inf)
    base = pl.program_id(0) * NPAD
    col128 = jax.lax.broadcasted_iota(jnp.int32, (NPAD, 128), 1)
    idxm = jnp.zeros((NPAD, 128), jnp.int32)
    sel0 = None
    for t in range(K):
        rowmin = jnp.min(dist, axis=1, keepdims=True)
        cand = jnp.where(dist == rowmin, col, NPAD)
        sel = jnp.min(cand, axis=1, keepdims=True)
        idxm = idxm + jnp.where(col128 == t, sel + base, 0)
        if sel0 is None:
            sel0 = sel
        dist = jnp.where(col == sel, jnp.inf, dist)
    idxm = jnp.where((col128 >= K) & (col128 < 16), sel0 + base, idxm)
    idx_ref[0] = idxm


def _vig_b_kernel(feat_ref, mr_ref, wgt_ref, wgb_ref, bg_ref, out_ref):
    feat = feat_ref[0]
    fb = feat.astype(jnp.bfloat16)
    hh = jnp.dot(fb, wgt_ref[...], preferred_element_type=jnp.float32)
    hh = hh + jnp.dot(mr_ref[0].astype(jnp.bfloat16), wgb_ref[...],
                      preferred_element_type=jnp.float32)
    hh = hh + bg_ref[...]
    out_ref[0] = feat + jnp.maximum(hh, 0.0)


NTILES = 32
CH = 4


def _sc_gather_kernel(feat_hbm, idx_hbm, mr_hbm, idxv, fbuf, gbuf, obuf, sem):
    nc = 2
    wid = jax.lax.axis_index("s") * nc + jax.lax.axis_index("c")
    nrows = feat_hbm.shape[0]
    rows_per_tile = nrows // NTILES
    tile_base = wid * rows_per_tile
    nchunks = rows_per_tile // CH

    def fire(slot, g):
        cbase = tile_base + g * CH
        pltpu.sync_copy(idx_hbm.at[pl.ds(cbase, CH)], idxv.at[slot])
        pltpu.sync_copy(feat_hbm.at[pl.ds(cbase, CH)], fbuf.at[slot])
        for t in range(CH):
            pltpu.async_copy(
                feat_hbm.at[idxv.at[slot, t, pl.ds(0, 16)]],
                gbuf.at[slot, t], sem)

    fire(0, 0)

    def chunk(g, carry):
        slot = jax.lax.rem(g, 2)
        nslot = jax.lax.rem(g + 1, 2)

        @pl.when(g + 1 < nchunks)
        def _prefetch():
            fire(nslot, g + 1)

        for t in range(CH):
            pltpu.make_async_copy(
                feat_hbm.at[idxv.at[slot, t, pl.ds(0, 16)]],
                gbuf.at[slot, t], sem).wait()
        def tbody(t, tc):
            for dd in range(D // 16):
                sl = pl.ds(dd * 16, 16)
                m = gbuf[slot, t, 0, sl]
                for r in range(1, 16):
                    m = jnp.maximum(m, gbuf[slot, t, r, sl])
                obuf[t, sl] = m - fbuf[slot, t, sl]
            return tc
        jax.lax.fori_loop(0, CH, tbody, 0)
        cbase = tile_base + g * CH
        pltpu.sync_copy(obuf, mr_hbm.at[pl.ds(cbase, CH)])
        return carry

    jax.lax.fori_loop(0, nchunks, chunk, 0)


def kernel(x, W0, b0, W1, b1, We, be, Wg, bg):
    band0 = _make_band(W0)
    band1 = _make_band(W1)

    amp_src, p = pl.pallas_call(
        _conv_patch_kernel,
        grid=(B,),
        in_specs=[
            pl.BlockSpec(memory_space=pltpu.SMEM),
            pl.BlockSpec(memory_space=pltpu.SMEM),
            pl.BlockSpec((C * 3, H + 2, C * H), lambda i: (0, 0, 0)),
            pl.BlockSpec((C * 3, H + 2, C * H), lambda i: (0, 0, 0)),
            pl.BlockSpec((1, C, H, H), lambda i: (i, 0, 0, 0)),
        ],
        out_specs=[
            pl.BlockSpec((1, C, H, H), lambda i: (i, 0, 0, 0)),
            pl.BlockSpec((1, NPAD, D), lambda i: (i, 0, 0)),
        ],
        out_shape=[
            jax.ShapeDtypeStruct((B, C, H, H), jnp.float32),
            jax.ShapeDtypeStruct((B, NPAD, D), jnp.float32),
        ],
        scratch_shapes=[
            pltpu.VMEM((C, H + 2, H + 2), jnp.float32),
            pltpu.VMEM((C, H + 2, H + 2), jnp.float32),
        ],
    )(b0, b1, band0, band1, x)

    web = We.astype(jnp.bfloat16)
    wgtb = Wg[:D].astype(jnp.bfloat16)
    wgbb = Wg[D:].astype(jnp.bfloat16)
    be2 = be.reshape(1, D)
    bg2 = bg.reshape(1, D)

    HB = B
    HROWS = HB * NPAD
    sc_gather = functools.partial(
        pl.kernel,
        mesh=plsc.VectorSubcoreMesh(core_axis_name="c", subcore_axis_name="s"),
        out_type=jax.ShapeDtypeStruct((HROWS, D), jnp.float32),
        scratch_types=[
            pltpu.VMEM((2, CH, 128), jnp.int32),
            pltpu.VMEM((2, CH, D), jnp.float32),
            pltpu.VMEM((2, CH, 16, D), jnp.float32),
            pltpu.VMEM((CH, D), jnp.float32),
            pltpu.SemaphoreType.DMA,
        ],
    )(_sc_gather_kernel)

    feat, idx = pl.pallas_call(
        _vig_a_kernel,
        grid=(HB,),
        in_specs=[
            pl.BlockSpec((1, NPAD, D), lambda i: (i, 0, 0)),
            pl.BlockSpec((D, D), lambda i: (0, 0)),
            pl.BlockSpec((1, D), lambda i: (0, 0)),
        ],
        out_specs=[
            pl.BlockSpec((1, NPAD, D), lambda i: (i, 0, 0)),
            pl.BlockSpec((1, NPAD, 128), lambda i: (i, 0, 0)),
        ],
        out_shape=[
            jax.ShapeDtypeStruct((HB, NPAD, D), jnp.float32),
            jax.ShapeDtypeStruct((HB, NPAD, 128), jnp.int32),
        ],
    )(p, web, be2)
    mr = sc_gather(feat.reshape(HROWS, D), idx.reshape(HROWS, 128))
    mr = mr.reshape(HB, NPAD, D)
    out = pl.pallas_call(
        _vig_b_kernel,
        grid=(HB,),
        in_specs=[
            pl.BlockSpec((1, NPAD, D), lambda i: (i, 0, 0)),
            pl.BlockSpec((1, NPAD, D), lambda i: (i, 0, 0)),
            pl.BlockSpec((D, D), lambda i: (0, 0)),
            pl.BlockSpec((D, D), lambda i: (0, 0)),
            pl.BlockSpec((1, D), lambda i: (0, 0)),
        ],
        out_specs=pl.BlockSpec((1, NPAD, D), lambda i: (i, 0, 0)),
        out_shape=jax.ShapeDtypeStruct((HB, NPAD, D), jnp.float32),
    )(feat, mr, wgtb, wgbb, bg2)

    amp_low = out[:, :N, :]
    return (amp_src, amp_low)


# speedup vs baseline: 1.0641x; 1.0632x over previous
import functools

import jax
import jax.numpy as jnp
from jax.experimental import pallas as pl
from jax.experimental.pallas import tpu as pltpu
from jax.experimental.pallas import tpu_sc as plsc

B = 8
C = 3
H = 352
N = 484
NPAD = 512
D = 768
K = 9
NEG = -3e38


def _band_matmul(src_b, band_ref):
    acc = jnp.zeros((H, C * H), jnp.float32)
    for ci in range(C):
        for dh in range(3):
            lhs = src_b[ci, dh:dh + H, :]
            rhs = band_ref[ci * 3 + dh]
            acc = acc + jnp.dot(lhs, rhs, preferred_element_type=jnp.float32)
    return acc


def _conv_patch_kernel(b0_ref, b1_ref, band0_ref, band1_ref, x_ref, amp_ref,
                       p_ref, xs, hs):
    @pl.when(pl.program_id(0) == 0)
    def _init():
        xs[...] = jnp.zeros_like(xs)
        hs[...] = jnp.zeros_like(hs)

    xs[:, 1:H + 1, 1:H + 1] = x_ref[0]
    xp = xs[...]
    y0 = _band_matmul(xp.astype(jnp.bfloat16), band0_ref)
    for co in range(C):
        hco = jnp.maximum(y0[:, co * H:(co + 1) * H] + b0_ref[co], 0.0)
        hs[co, 1:H + 1, 1:H + 1] = hco
    y1 = _band_matmul(hs[...].astype(jnp.bfloat16), band1_ref)
    prs = []
    for co in range(C):
        pr = y1[:, co * H:(co + 1) * H] + b1_ref[co]
        amp_ref[0, co] = pr * xp[co, 1:H + 1, 1:H + 1]
        prs.append(pr)
    v = jnp.stack(prs)
    v = v.reshape(C, 22, 16, 22, 16)
    v = jnp.transpose(v, (1, 3, 0, 2, 4))
    v = v.reshape(N, D)
    p_ref[0] = jnp.concatenate([v, jnp.zeros((NPAD - N, D), jnp.float32)], axis=0)


def _make_band(W):
    jp = jnp.arange(H + 2)
    j = jnp.arange(H)
    diff = jp[:, None] - j[None, :]
    vals = jnp.zeros((C, C, 3, H + 2, H), jnp.float32)
    for dw in range(3):
        mask = (diff == dw).astype(jnp.float32)
        vals = vals + W[:, :, :, dw][..., None, None] * mask
    vals = vals.transpose(1, 2, 3, 0, 4)
    return vals.reshape(C * 3, H + 2, C * H).astype(jnp.bfloat16)


def _vig_a_kernel(p_ref, we_ref, be_ref, feat_ref, idx_ref):
    pb = p_ref[0].astype(jnp.bfloat16)
    feat = jnp.dot(pb, we_ref[...], preferred_element_type=jnp.float32)
    feat = feat + be_ref[...]
    feat_ref[0] = feat
    sq = jnp.sum(feat * feat, axis=1, keepdims=True)
    fb = feat.astype(jnp.bfloat16)
    gram = jax.lax.dot_general(fb, fb, (((1,), (1,)), ((), ())),
                               preferred_element_type=jnp.float32)
    dist = sq + sq.T - 2.0 * gram
    col = jax.lax.broadcasted_iota(jnp.int32, (NPAD, NPAD), 1)
    dist = jnp.where(col < N, dist, jnp.inf)
    base = pl.program_id(0) * NPAD
    col128 = jax.lax.broadcasted_iota(jnp.int32, (NPAD, 128), 1)
    idxm = jnp.zeros((NPAD, 128), jnp.int32)
    sel0 = None
    for t in range(K):
        rowmin = jnp.min(dist, axis=1, keepdims=True)
        cand = jnp.where(dist == rowmin, col, NPAD)
        sel = jnp.min(cand, axis=1, keepdims=True)
        part = col128 - 6 * t
        idxm = idxm + jnp.where((part >= 0) & (part < 6),
                                (sel + base) * 6 + part, 0)
        if sel0 is None:
            sel0 = sel
        dist = jnp.where(col == sel, jnp.inf, dist)
    idxm = jnp.where((col128 >= 6 * K) & (col128 < 64), (sel0 + base) * 6, idxm)
    idx_ref[0] = idxm


def _vig_b_kernel(feat_ref, mr_ref, wgt_ref, wgb_ref, bg_ref, out_ref):
    feat = feat_ref[0]
    fb = feat.astype(jnp.bfloat16)
    hh = jnp.dot(fb, wgt_ref[...], preferred_element_type=jnp.float32)
    hh = hh + jnp.dot(mr_ref[0].astype(jnp.bfloat16), wgb_ref[...],
                      preferred_element_type=jnp.float32)
    hh = hh + bg_ref[...]
    out_ref[0] = feat + jnp.maximum(hh, 0.0)


NTILES = 32
CH = 4


SUB = 128
NSUB = D // SUB
GL = 64


def _sc_gather_kernel(feat_hbm, idx_hbm, mr_hbm, idxv, fbuf, gbuf, obuf, sem):
    nc = 2
    wid = jax.lax.axis_index("s") * nc + jax.lax.axis_index("c")
    nrows = feat_hbm.shape[0] // NSUB
    rows_per_tile = nrows // NTILES
    tile_base = wid * rows_per_tile
    nchunks = rows_per_tile // CH

    def fire(slot, g):
        cbase = tile_base + g * CH
        pltpu.sync_copy(idx_hbm.at[pl.ds(cbase, CH)], idxv.at[slot])
        pltpu.sync_copy(feat_hbm.at[pl.ds(cbase * NSUB, CH * NSUB)], fbuf.at[slot])
        for t in range(CH):
            pltpu.async_copy(
                feat_hbm.at[idxv.at[slot, t, pl.ds(0, GL)]],
                gbuf.at[slot, t], sem)

    fire(0, 0)

    def chunk(g, carry):
        slot = jax.lax.rem(g, 2)
        nslot = jax.lax.rem(g + 1, 2)

        @pl.when(g + 1 < nchunks)
        def _prefetch():
            fire(nslot, g + 1)

        for t in range(CH):
            pltpu.make_async_copy(
                feat_hbm.at[idxv.at[slot, t, pl.ds(0, GL)]],
                gbuf.at[slot, t], sem).wait()
        def tbody(t, tc):
            for part in range(NSUB):
                for w3 in range(SUB // 16):
                    sl = pl.ds(w3 * 16, 16)
                    m = gbuf[slot, t, part, sl]
                    for r in range(1, K):
                        m = jnp.maximum(m, gbuf[slot, t, r * NSUB + part, sl])
                    obuf[t, pl.ds(part * SUB + w3 * 16, 16)] = (
                        m - fbuf[slot, t * NSUB + part, sl])
            return tc
        jax.lax.fori_loop(0, CH, tbody, 0)
        cbase = tile_base + g * CH
        pltpu.sync_copy(obuf, mr_hbm.at[pl.ds(cbase, CH)])
        return carry

    jax.lax.fori_loop(0, nchunks, chunk, 0)


def kernel(x, W0, b0, W1, b1, We, be, Wg, bg):
    band0 = _make_band(W0)
    band1 = _make_band(W1)

    amp_src, p = pl.pallas_call(
        _conv_patch_kernel,
        grid=(B,),
        in_specs=[
            pl.BlockSpec(memory_space=pltpu.SMEM),
            pl.BlockSpec(memory_space=pltpu.SMEM),
            pl.BlockSpec((C * 3, H + 2, C * H), lambda i: (0, 0, 0)),
            pl.BlockSpec((C * 3, H + 2, C * H), lambda i: (0, 0, 0)),
            pl.BlockSpec((1, C, H, H), lambda i: (i, 0, 0, 0)),
        ],
        out_specs=[
            pl.BlockSpec((1, C, H, H), lambda i: (i, 0, 0, 0)),
            pl.BlockSpec((1, NPAD, D), lambda i: (i, 0, 0)),
        ],
        out_shape=[
            jax.ShapeDtypeStruct((B, C, H, H), jnp.float32),
            jax.ShapeDtypeStruct((B, NPAD, D), jnp.float32),
        ],
        scratch_shapes=[
            pltpu.VMEM((C, H + 2, H + 2), jnp.float32),
            pltpu.VMEM((C, H + 2, H + 2), jnp.float32),
        ],
    )(b0, b1, band0, band1, x)

    web = We.astype(jnp.bfloat16)
    wgtb = Wg[:D].astype(jnp.bfloat16)
    wgbb = Wg[D:].astype(jnp.bfloat16)
    be2 = be.reshape(1, D)
    bg2 = bg.reshape(1, D)

    HB = B
    HROWS = HB * NPAD
    sc_gather = functools.partial(
        pl.kernel,
        mesh=plsc.VectorSubcoreMesh(core_axis_name="c", subcore_axis_name="s"),
        out_type=jax.ShapeDtypeStruct((HROWS, D), jnp.float32),
        scratch_types=[
            pltpu.VMEM((2, CH, 128), jnp.int32),
            pltpu.VMEM((2, CH * NSUB, SUB), jnp.float32),
            pltpu.VMEM((2, CH, GL, SUB), jnp.float32),
            pltpu.VMEM((CH, D), jnp.float32),
            pltpu.SemaphoreType.DMA,
        ],
    )(_sc_gather_kernel)

    feat, idx = pl.pallas_call(
        _vig_a_kernel,
        grid=(HB,),
        in_specs=[
            pl.BlockSpec((1, NPAD, D), lambda i: (i, 0, 0)),
            pl.BlockSpec((D, D), lambda i: (0, 0)),
            pl.BlockSpec((1, D), lambda i: (0, 0)),
        ],
        out_specs=[
            pl.BlockSpec((1, NPAD, D), lambda i: (i, 0, 0)),
            pl.BlockSpec((1, NPAD, 128), lambda i: (i, 0, 0)),
        ],
        out_shape=[
            jax.ShapeDtypeStruct((HB, NPAD, D), jnp.float32),
            jax.ShapeDtypeStruct((HB, NPAD, 128), jnp.int32),
        ],
    )(p, web, be2)
    mr = sc_gather(feat.reshape(HROWS * NSUB, SUB), idx.reshape(HROWS, 128))
    mr = mr.reshape(HB, NPAD, D)
    out = pl.pallas_call(
        _vig_b_kernel,
        grid=(HB,),
        in_specs=[
            pl.BlockSpec((1, NPAD, D), lambda i: (i, 0, 0)),
            pl.BlockSpec((1, NPAD, D), lambda i: (i, 0, 0)),
            pl.BlockSpec((D, D), lambda i: (0, 0)),
            pl.BlockSpec((D, D), lambda i: (0, 0)),
            pl.BlockSpec((1, D), lambda i: (0, 0)),
        ],
        out_specs=pl.BlockSpec((1, NPAD, D), lambda i: (i, 0, 0)),
        out_shape=jax.ShapeDtypeStruct((HB, NPAD, D), jnp.float32),
    )(feat, mr, wgtb, wgbb, bg2)

    amp_low = out[:, :N, :]
    return (amp_src, amp_low)
